# baseline (speedup 1.0000x reference)
import functools

import jax
import jax.numpy as jnp
from jax.experimental import pallas as pl
from jax.experimental.pallas import tpu as pltpu

B = 16
LX = 1024
LC = 1024
LT = 2048
D = 64
PAD = -999.0
RPS = 4
NSTEPS = B // RPS


def _one_row(lx, lc, beta, xl_blk, trow):
    r = (lx - 1) % 8
    row_ids = jax.lax.broadcasted_iota(jnp.int32, (8, D), 0)
    col_ids = jax.lax.broadcasted_iota(jnp.int32, (8, D), 1)
    x_last = jnp.sum(jnp.where((row_ids == r) & (col_ids == 0), xl_blk, 0.0))

    t0 = trow[0, 0]
    rot = pltpu.roll(trow, LT - lx, 1)
    t_last = rot[0, LT - 1] - t0
    slope = (x_last - beta) / t_last

    fut = rot[:, :LC] - t0
    pred = slope * fut + beta

    pos = jax.lax.broadcasted_iota(jnp.int32, (LC, 1), 0)
    col = jnp.where(pos < lc, pred.reshape(LC, 1), PAD)

    d_ids = jax.lax.broadcasted_iota(jnp.int32, (LC, D), 1)
    return jnp.where(d_ids == 0, col, PAD)


def _xl_copy(x_hbm, lx_ref, xl_buf, sems, gg, slot):
    copies = []
    for k in range(RPS):
        i = RPS * gg + k
        base = ((lx_ref[i] - 1) // 8) * 8
        copies.append(pltpu.make_async_copy(
            x_hbm.at[i, pl.ds(base, 8), :], xl_buf.at[slot, k],
            sems.at[slot, k]))
    return copies


def _row_kernel(lx_ref, lc_ref, x0_ref, x_hbm, t_ref, o_ref, xl_buf, sems):
    g = pl.program_id(0)
    slot = jax.lax.rem(g, 2)

    @pl.when(g == 0)
    def _prime():
        for c in _xl_copy(x_hbm, lx_ref, xl_buf, sems, 0, 0):
            c.start()

    @pl.when(g + 1 < NSTEPS)
    def _prefetch():
        for c in _xl_copy(x_hbm, lx_ref, xl_buf, sems, g + 1,
                          jax.lax.rem(g + 1, 2)):
            c.start()

    for c in _xl_copy(x_hbm, lx_ref, xl_buf, sems, g, slot):
        c.wait()

    for k in range(RPS):
        i = RPS * g + k
        o_ref[k] = _one_row(lx_ref[i], lc_ref[i], x0_ref[i, 0, 0],
                            xl_buf[slot, k], t_ref[pl.ds(i, 1)])


@functools.partial(jax.jit, static_argnames=("interpret",))
def _run(x, time, len_x, len_context, interpret=False):
    grid_spec = pltpu.PrefetchScalarGridSpec(
        num_scalar_prefetch=2,
        grid=(NSTEPS,),
        in_specs=[
            pl.BlockSpec((B, 8, D), lambda g, lx, lc: (0, 0, 0)),
            pl.BlockSpec(memory_space=pltpu.MemorySpace.HBM),
            pl.BlockSpec((B, LT), lambda g, lx, lc: (0, 0)),
        ],
        out_specs=pl.BlockSpec((RPS, LC, D), lambda g, lx, lc: (g, 0, 0)),
        scratch_shapes=[
            pltpu.VMEM((2, RPS, 8, D), jnp.float32),
            pltpu.SemaphoreType.DMA((2, RPS)),
        ],
    )
    return pl.pallas_call(
        _row_kernel,
        grid_spec=grid_spec,
        out_shape=jax.ShapeDtypeStruct((B, LC, D), jnp.float32),
        interpret=interpret,
    )(len_x, len_context, x, x, time)


def kernel(x, time, context, len_x, len_context):
    return _run(x, time, len_x, len_context)


# speedup vs baseline: 1.0114x; 1.0114x over previous
import functools

import jax
import jax.numpy as jnp
from jax.experimental import pallas as pl
from jax.experimental.pallas import tpu as pltpu

B = 16
LX = 1024
LC = 1024
LT = 2048
D = 64
PAD = -999.0
RPS = 2


def _one_row(lx, lc, beta, xl_blk, trow):
    r = (lx - 1) % 8
    row_ids = jax.lax.broadcasted_iota(jnp.int32, (8, D), 0)
    col_ids = jax.lax.broadcasted_iota(jnp.int32, (8, D), 1)
    x_last = jnp.sum(jnp.where((row_ids == r) & (col_ids == 0), xl_blk, 0.0))

    t0 = trow[0, 0]
    rot = pltpu.roll(trow, LT - lx, 1)
    t_last = rot[0, LT - 1] - t0
    slope = (x_last - beta) / t_last

    fut = rot[:, :LC] - t0
    pred = slope * fut + beta

    pos = jax.lax.broadcasted_iota(jnp.int32, (LC, 1), 0)
    col = jnp.where(pos < lc, pred.reshape(LC, 1), PAD)

    d_ids = jax.lax.broadcasted_iota(jnp.int32, (LC, D), 1)
    return jnp.where(d_ids == 0, col, PAD)


def _row_kernel(lx_ref, lc_ref, *refs):
    x0_ref = refs[0]
    xl_refs = refs[1: 1 + RPS]
    t_ref = refs[1 + RPS]
    o_ref = refs[2 + RPS]
    g = pl.program_id(0)
    for k in range(RPS):
        i = RPS * g + k
        o_ref[k] = _one_row(lx_ref[i], lc_ref[i], x0_ref[i, 0, 0],
                            xl_refs[k][0], t_ref[pl.ds(i, 1)])


def _x_specs():
    specs = [pl.BlockSpec((B, 8, D), lambda g, lx, lc: (0, 0, 0))]
    for k in range(RPS):
        specs.append(
            pl.BlockSpec(
                (1, 8, D),
                lambda g, lx, lc, k=k:
                (RPS * g + k, (lx[RPS * g + k] - 1) // 8, 0)))
    return specs


@functools.partial(jax.jit, static_argnames=("interpret",))
def _run(x, time, len_x, len_context, interpret=False):
    grid_spec = pltpu.PrefetchScalarGridSpec(
        num_scalar_prefetch=2,
        grid=(B // RPS,),
        in_specs=_x_specs() + [pl.BlockSpec((B, LT), lambda g, lx, lc: (0, 0))],
        out_specs=pl.BlockSpec((RPS, LC, D), lambda g, lx, lc: (g, 0, 0)),
    )
    return pl.pallas_call(
        _row_kernel,
        grid_spec=grid_spec,
        out_shape=jax.ShapeDtypeStruct((B, LC, D), jnp.float32),
        interpret=interpret,
    )(len_x, len_context, *([x] * (1 + RPS)), time)


def kernel(x, time, context, len_x, len_context):
    return _run(x, time, len_x, len_context)


# speedup vs baseline: 1.0232x; 1.0116x over previous
import functools

import jax
import jax.numpy as jnp
from jax.experimental import pallas as pl
from jax.experimental.pallas import tpu as pltpu

B = 16
LX = 1024
LC = 1024
LT = 2048
D = 64
PAD = -999.0
RPS = 4


def _one_row(lx, lc, beta, xl_blk, trow):
    r = (lx - 1) % 8
    row_ids = jax.lax.broadcasted_iota(jnp.int32, (8, D), 0)
    col_ids = jax.lax.broadcasted_iota(jnp.int32, (8, D), 1)
    x_last = jnp.sum(jnp.where((row_ids == r) & (col_ids == 0), xl_blk, 0.0))

    t0 = trow[0, 0]
    rot = pltpu.roll(trow, LT - lx, 1)
    t_last = rot[0, LT - 1] - t0
    slope = (x_last - beta) / t_last

    fut = rot[:, :LC] - t0
    pred = slope * fut + beta

    pos = jax.lax.broadcasted_iota(jnp.int32, (LC, 1), 0)
    col = jnp.where(pos < lc, pred.reshape(LC, 1), PAD)

    d_ids = jax.lax.broadcasted_iota(jnp.int32, (LC, D), 1)
    return jnp.where(d_ids == 0, col, PAD)


def _row_kernel(lx_ref, lc_ref, *refs):
    x0_ref = refs[0]
    xl_refs = refs[1: 1 + RPS]
    t_ref = refs[1 + RPS]
    o_ref = refs[2 + RPS]
    g = pl.program_id(0)
    for k in range(RPS):
        i = RPS * g + k
        o_ref[k] = _one_row(lx_ref[i], lc_ref[i], x0_ref[i, 0, 0],
                            xl_refs[k][0], t_ref[pl.ds(i, 1)])


def _x_specs():
    specs = [pl.BlockSpec((B, 8, D), lambda g, lx, lc: (0, 0, 0))]
    for k in range(RPS):
        specs.append(
            pl.BlockSpec(
                (1, 8, D),
                lambda g, lx, lc, k=k:
                (RPS * g + k, (lx[RPS * g + k] - 1) // 8, 0)))
    return specs


@functools.partial(jax.jit, static_argnames=("interpret",))
def _run(x, time, len_x, len_context, interpret=False):
    grid_spec = pltpu.PrefetchScalarGridSpec(
        num_scalar_prefetch=2,
        grid=(B // RPS,),
        in_specs=_x_specs() + [pl.BlockSpec((B, LT), lambda g, lx, lc: (0, 0))],
        out_specs=pl.BlockSpec((RPS, LC, D), lambda g, lx, lc: (g, 0, 0)),
    )
    return pl.pallas_call(
        _row_kernel,
        grid_spec=grid_spec,
        out_shape=jax.ShapeDtypeStruct((B, LC, D), jnp.float32),
        interpret=interpret,
    )(len_x, len_context, *([x] * (1 + RPS)), time)


def kernel(x, time, context, len_x, len_context):
    return _run(x, time, len_x, len_context)
